# baseline (speedup 1.0000x reference)
import functools
import jax
import jax.numpy as jnp
from jax import lax
from jax.experimental import pallas as pl
from jax.experimental.pallas import tpu as pltpu
from jax.experimental.pallas import tpu_sc as plsc

_N = 4 * 2048 * 4096
_NW = 32
_PER_W = _N // _NW
_C = 16384
_NCH = _PER_W // _C
_IT = _C // 64


def _sc_masked_mse(yp_hbm, yt_hbm, m32_hbm, out_s_hbm, out_c_hbm,
                   ypb, ytb, mb, accv, cntv,
                   s_yp0, s_yp1, s_yt0, s_yt1, s_m0, s_m1):
    wid = lax.axis_index("s") * 2 + lax.axis_index("c")
    base = wid * _PER_W

    sems = ((s_yp0, s_yt0, s_m0), (s_yp1, s_yt1, s_m1))

    def start(c, b):
        off = pl.multiple_of(base + c * _C, 4096)
        offm = pl.multiple_of((base + c * _C) // 4, 1024)
        pltpu.async_copy(yp_hbm.at[pl.ds(off, _C)], ypb.at[b], sems[b][0])
        pltpu.async_copy(yt_hbm.at[pl.ds(off, _C)], ytb.at[b], sems[b][1])
        pltpu.async_copy(m32_hbm.at[pl.ds(offm, _C // 4)], mb.at[b],
                         sems[b][2])

    def wait(c, b):
        off = pl.multiple_of(base + c * _C, 4096)
        offm = pl.multiple_of((base + c * _C) // 4, 1024)
        pltpu.make_async_copy(yp_hbm.at[pl.ds(off, _C)], ypb.at[b],
                              sems[b][0]).wait()
        pltpu.make_async_copy(yt_hbm.at[pl.ds(off, _C)], ytb.at[b],
                              sems[b][1]).wait()
        pltpu.make_async_copy(m32_hbm.at[pl.ds(offm, _C // 4)], mb.at[b],
                              sems[b][2]).wait()

    start(0, 0)
    start(1, 1)

    k = lax.iota(jnp.int32, 16)
    sh = (k & 3) << 3
    gidx = k >> 2

    def chunk_compute(b, accs):
        def body(i, carry):
            a0, a1, a2, a3, c0, c1, c2, c3 = carry
            w = i * 16
            e = i * 64
            m32 = mb[b, pl.ds(w, 16)]

            def lane_mask(t):
                g = m32.at[gidx + 4 * t].get(mode="promise_in_bounds")
                return ((g >> sh) & 1).astype(jnp.float32)

            def term(t):
                d = (ypb[b, pl.ds(e + 16 * t, 16)]
                     - ytb[b, pl.ds(e + 16 * t, 16)])
                mf = lane_mask(t)
                return d * d * mf, mf

            t0, m0 = term(0)
            t1, m1 = term(1)
            t2, m2 = term(2)
            t3, m3 = term(3)
            return (a0 + t0, a1 + t1, a2 + t2, a3 + t3,
                    c0 + m0, c1 + m1, c2 + m2, c3 + m3)

        return lax.fori_loop(0, _IT, body, accs, unroll=4)

    z = jnp.zeros((16,), jnp.float32)
    accs = (z, z, z, z, z, z, z, z)

    def outer(go, accs):
        for b in (0, 1):
            c = 2 * go + b
            wait(c, b)
            @pl.when(c + 2 < _NCH)
            def _():
                start(c + 2, b)
            accs = chunk_compute(b, accs)
        return accs

    accs = lax.fori_loop(0, _NCH // 2, outer, accs)
    a0, a1, a2, a3, c0, c1, c2, c3 = accs
    accv[...] = (a0 + a1) + (a2 + a3)
    cntv[...] = (c0 + c1) + (c2 + c3)
    pltpu.sync_copy(accv, out_s_hbm.at[wid])
    pltpu.sync_copy(cntv, out_c_hbm.at[wid])


def _sc_partials(yp_flat, yt_flat, m32_flat):
    mesh = plsc.VectorSubcoreMesh(core_axis_name="c", subcore_axis_name="s")
    f = pl.kernel(
        _sc_masked_mse,
        mesh=mesh,
        out_type=[
            jax.ShapeDtypeStruct((_NW, 16), jnp.float32),
            jax.ShapeDtypeStruct((_NW, 16), jnp.float32),
        ],
        scratch_types=[
            pltpu.VMEM((2, _C), jnp.float32),
            pltpu.VMEM((2, _C), jnp.float32),
            pltpu.VMEM((2, _C // 4), jnp.int32),
            pltpu.VMEM((16,), jnp.float32),
            pltpu.VMEM((16,), jnp.float32),
            pltpu.SemaphoreType.DMA,
            pltpu.SemaphoreType.DMA,
            pltpu.SemaphoreType.DMA,
            pltpu.SemaphoreType.DMA,
            pltpu.SemaphoreType.DMA,
            pltpu.SemaphoreType.DMA,
        ],
    )
    return f(yp_flat, yt_flat, m32_flat)


def kernel(y_pred, y_true, mask):
    yp = y_pred.reshape(_N)
    yt = y_true.reshape(_N)
    m8 = mask.reshape(_N).view(jnp.uint8)
    m32 = lax.bitcast_convert_type(m8.reshape(_N // 4, 4), jnp.int32)
    s, c = _sc_partials(yp, yt, m32)
    return jnp.sum(s) / jnp.sum(c)


# speedup vs baseline: 14.5990x; 14.5990x over previous
import functools
import jax
import jax.numpy as jnp
from jax import lax
from jax.experimental import pallas as pl
from jax.experimental.pallas import tpu as pltpu
from jax.experimental.pallas import tpu_sc as plsc

_N = 4 * 2048 * 4096
_NW = 32
_PER_W = _N // _NW
_C = 16384
_NCH = _PER_W // _C
_IT = _C // 64
_MCOLS = 4096
_MR = _C // _MCOLS


def _sc_masked_mse(yp_hbm, yt_hbm, m8_hbm, out_s_hbm, out_c_hbm,
                   ypb, ytb, mb, accv, cntv,
                   s_yp0, s_yp1, s_yt0, s_yt1, s_m0, s_m1):
    wid = lax.axis_index("s") * 2 + lax.axis_index("c")
    base = wid * _PER_W

    sems = ((s_yp0, s_yt0, s_m0), (s_yp1, s_yt1, s_m1))

    def start(c, b):
        off = pl.multiple_of(base + c * _C, 4096)
        mrow = pl.multiple_of((base + c * _C) // _MCOLS, 4)
        pltpu.async_copy(yp_hbm.at[pl.ds(off, _C)], ypb.at[b], sems[b][0])
        pltpu.async_copy(yt_hbm.at[pl.ds(off, _C)], ytb.at[b], sems[b][1])
        pltpu.async_copy(m8_hbm.at[pl.ds(mrow, _MR)],
                         mb.at[pl.ds(b * _MR, _MR)], sems[b][2])

    def wait(c, b):
        off = pl.multiple_of(base + c * _C, 4096)
        mrow = pl.multiple_of((base + c * _C) // _MCOLS, 4)
        pltpu.make_async_copy(yp_hbm.at[pl.ds(off, _C)], ypb.at[b],
                              sems[b][0]).wait()
        pltpu.make_async_copy(yt_hbm.at[pl.ds(off, _C)], ytb.at[b],
                              sems[b][1]).wait()
        pltpu.make_async_copy(m8_hbm.at[pl.ds(mrow, _MR)],
                              mb.at[pl.ds(b * _MR, _MR)], sems[b][2]).wait()

    start(0, 0)
    start(1, 1)

    mbi = mb.bitcast(jnp.int32)

    k = lax.iota(jnp.int32, 16)
    sh = (k & 3) << 3
    gidx = k >> 2

    def chunk_compute(b, accs):
        def body(i, carry):
            a0, a1, a2, a3, c0, c1, c2, c3 = carry
            e = i * 64
            m32 = mbi[b, pl.ds(pl.multiple_of(e // 4, 16), 16)]

            def lane_mask(t):
                g = m32.at[gidx + 4 * t].get(mode="promise_in_bounds")
                return ((g >> sh) & 1).astype(jnp.float32)

            def term(t):
                o = pl.multiple_of(e + 16 * t, 16)
                d = ypb[b, pl.ds(o, 16)] - ytb[b, pl.ds(o, 16)]
                mf = lane_mask(t)
                return d * d * mf, mf

            t0, m0 = term(0)
            t1, m1 = term(1)
            t2, m2 = term(2)
            t3, m3 = term(3)
            return (a0 + t0, a1 + t1, a2 + t2, a3 + t3,
                    c0 + m0, c1 + m1, c2 + m2, c3 + m3)

        return lax.fori_loop(0, _IT, body, accs, unroll=4)

    z = jnp.zeros((16,), jnp.float32)
    accs = (z, z, z, z, z, z, z, z)

    def outer(go, accs):
        for b in (0, 1):
            c = 2 * go + b
            wait(c, b)
            @pl.when(c + 2 < _NCH)
            def _():
                start(c + 2, b)
            accs = chunk_compute(b, accs)
        return accs

    accs = lax.fori_loop(0, _NCH // 2, outer, accs)
    a0, a1, a2, a3, c0, c1, c2, c3 = accs
    accv[...] = (a0 + a1) + (a2 + a3)
    cntv[...] = (c0 + c1) + (c2 + c3)
    pltpu.sync_copy(accv, out_s_hbm.at[wid])
    pltpu.sync_copy(cntv, out_c_hbm.at[wid])


def _sc_partials(yp_flat, yt_flat, m8_flat):
    mesh = plsc.VectorSubcoreMesh(core_axis_name="c", subcore_axis_name="s")
    f = pl.kernel(
        _sc_masked_mse,
        mesh=mesh,
        out_type=[
            jax.ShapeDtypeStruct((_NW, 16), jnp.float32),
            jax.ShapeDtypeStruct((_NW, 16), jnp.float32),
        ],
        scratch_types=[
            pltpu.VMEM((2, _C), jnp.float32),
            pltpu.VMEM((2, _C), jnp.float32),
            pltpu.VMEM((2 * _MR, _MCOLS), jnp.uint8),
            pltpu.VMEM((16,), jnp.float32),
            pltpu.VMEM((16,), jnp.float32),
            pltpu.SemaphoreType.DMA,
            pltpu.SemaphoreType.DMA,
            pltpu.SemaphoreType.DMA,
            pltpu.SemaphoreType.DMA,
            pltpu.SemaphoreType.DMA,
            pltpu.SemaphoreType.DMA,
        ],
    )
    return f(yp_flat, yt_flat, m8_flat)


def kernel(y_pred, y_true, mask):
    yp = y_pred.reshape(_N)
    yt = y_true.reshape(_N)
    m8 = mask.reshape(_N // _MCOLS, _MCOLS).view(jnp.uint8)
    s, c = _sc_partials(yp, yt, m8)
    return jnp.sum(s) / jnp.sum(c)


# speedup vs baseline: 16.1406x; 1.1056x over previous
import jax
import jax.numpy as jnp
from jax import lax
from jax.experimental import pallas as pl
from jax.experimental.pallas import tpu as pltpu
from jax.experimental.pallas import tpu_sc as plsc

_ROWS = 8192
_COLS = 4096
_N = _ROWS * _COLS

_SC_ROWS = 2048
_TC_ROWS = _ROWS - _SC_ROWS
_BLOCK_ROWS = 256

_NW = 32
_SC_N = _SC_ROWS * _COLS
_PER_W = _SC_N // _NW
_C = 16384
_NCH = _PER_W // _C
_IT = _C // 64
_MR = _C // _COLS


def _tc_kernel(yp_ref, yt_ref, m_ref, sum_ref, cnt_ref, acc_s, acc_c):
    i = pl.program_id(0)

    @pl.when(i == 0)
    def _init():
        acc_s[...] = jnp.zeros_like(acc_s)
        acc_c[...] = jnp.zeros_like(acc_c)

    d = yp_ref[...] - yt_ref[...]
    m = m_ref[...]
    sq = jnp.where(m, d * d, jnp.float32(0.0))
    c = m.astype(jnp.float32)
    ps = sq[0:8]
    pc = c[0:8]
    for k in range(1, _BLOCK_ROWS // 8):
        ps = ps + sq[8 * k:8 * k + 8]
        pc = pc + c[8 * k:8 * k + 8]
    acc_s[...] += ps
    acc_c[...] += pc

    @pl.when(i == pl.num_programs(0) - 1)
    def _fini():
        sum_ref[0, 0] = jnp.sum(acc_s[...])
        cnt_ref[0, 0] = jnp.sum(acc_c[...])


def _tc_partials(yp, yt, m):
    grid = (_TC_ROWS // _BLOCK_ROWS,)
    in_spec = pl.BlockSpec((_BLOCK_ROWS, _COLS), lambda i: (i, 0))
    out_spec = pl.BlockSpec(memory_space=pltpu.SMEM)
    return pl.pallas_call(
        _tc_kernel,
        grid=grid,
        in_specs=[in_spec, in_spec, in_spec],
        out_specs=[out_spec, out_spec],
        out_shape=[
            jax.ShapeDtypeStruct((1, 1), jnp.float32),
            jax.ShapeDtypeStruct((1, 1), jnp.float32),
        ],
        scratch_shapes=[
            pltpu.VMEM((8, _COLS), jnp.float32),
            pltpu.VMEM((8, _COLS), jnp.float32),
        ],
    )(yp, yt, m)


def _sc_masked_mse(yp_hbm, yt_hbm, m8_hbm, out_s_hbm, out_c_hbm,
                   ypb, ytb, mb, accv, cntv,
                   s_yp0, s_yp1, s_yt0, s_yt1, s_m0, s_m1):
    wid = lax.axis_index("s") * 2 + lax.axis_index("c")
    base = wid * _PER_W

    sems = ((s_yp0, s_yt0, s_m0), (s_yp1, s_yt1, s_m1))

    def start(c, b):
        off = pl.multiple_of(base + c * _C, 4096)
        mrow = pl.multiple_of((base + c * _C) // _COLS, 4)
        pltpu.async_copy(yp_hbm.at[pl.ds(off, _C)], ypb.at[b], sems[b][0])
        pltpu.async_copy(yt_hbm.at[pl.ds(off, _C)], ytb.at[b], sems[b][1])
        pltpu.async_copy(m8_hbm.at[pl.ds(mrow, _MR)],
                         mb.at[pl.ds(b * _MR, _MR)], sems[b][2])

    def wait(c, b):
        off = pl.multiple_of(base + c * _C, 4096)
        mrow = pl.multiple_of((base + c * _C) // _COLS, 4)
        pltpu.make_async_copy(yp_hbm.at[pl.ds(off, _C)], ypb.at[b],
                              sems[b][0]).wait()
        pltpu.make_async_copy(yt_hbm.at[pl.ds(off, _C)], ytb.at[b],
                              sems[b][1]).wait()
        pltpu.make_async_copy(m8_hbm.at[pl.ds(mrow, _MR)],
                              mb.at[pl.ds(b * _MR, _MR)], sems[b][2]).wait()

    start(0, 0)
    start(1, 1)

    mbi = mb.bitcast(jnp.int32)

    k = lax.iota(jnp.int32, 16)
    sh = (k & 3) << 3
    gidx = k >> 2

    def chunk_compute(b, accs):
        def body(i, carry):
            a0, a1, a2, a3, c0, c1, c2, c3 = carry
            e = i * 64
            m32 = mbi[b, pl.ds(pl.multiple_of(e // 4, 16), 16)]

            def lane_mask(t):
                g = m32.at[gidx + 4 * t].get(mode="promise_in_bounds")
                return ((g >> sh) & 1).astype(jnp.float32)

            def term(t):
                o = pl.multiple_of(e + 16 * t, 16)
                d = ypb[b, pl.ds(o, 16)] - ytb[b, pl.ds(o, 16)]
                mf = lane_mask(t)
                return d * d * mf, mf

            t0, m0 = term(0)
            t1, m1 = term(1)
            t2, m2 = term(2)
            t3, m3 = term(3)
            return (a0 + t0, a1 + t1, a2 + t2, a3 + t3,
                    c0 + m0, c1 + m1, c2 + m2, c3 + m3)

        return lax.fori_loop(0, _IT, body, accs, unroll=4)

    z = jnp.zeros((16,), jnp.float32)
    accs = (z, z, z, z, z, z, z, z)

    def outer(go, accs):
        for b in (0, 1):
            c = 2 * go + b
            wait(c, b)

            @pl.when(c + 2 < _NCH)
            def _():
                start(c + 2, b)

            accs = chunk_compute(b, accs)
        return accs

    accs = lax.fori_loop(0, _NCH // 2, outer, accs)
    a0, a1, a2, a3, c0, c1, c2, c3 = accs
    accv[...] = (a0 + a1) + (a2 + a3)
    cntv[...] = (c0 + c1) + (c2 + c3)
    pltpu.sync_copy(accv, out_s_hbm.at[wid])
    pltpu.sync_copy(cntv, out_c_hbm.at[wid])


def _sc_partials(yp_flat, yt_flat, m8_2d):
    mesh = plsc.VectorSubcoreMesh(core_axis_name="c", subcore_axis_name="s")
    f = pl.kernel(
        _sc_masked_mse,
        mesh=mesh,
        out_type=[
            jax.ShapeDtypeStruct((_NW, 16), jnp.float32),
            jax.ShapeDtypeStruct((_NW, 16), jnp.float32),
        ],
        scratch_types=[
            pltpu.VMEM((2, _C), jnp.float32),
            pltpu.VMEM((2, _C), jnp.float32),
            pltpu.VMEM((2 * _MR, _COLS), jnp.uint8),
            pltpu.VMEM((16,), jnp.float32),
            pltpu.VMEM((16,), jnp.float32),
            pltpu.SemaphoreType.DMA,
            pltpu.SemaphoreType.DMA,
            pltpu.SemaphoreType.DMA,
            pltpu.SemaphoreType.DMA,
            pltpu.SemaphoreType.DMA,
            pltpu.SemaphoreType.DMA,
        ],
    )
    return f(yp_flat, yt_flat, m8_2d)


def kernel(y_pred, y_true, mask):
    yp = y_pred.reshape(_ROWS, _COLS)
    yt = y_true.reshape(_ROWS, _COLS)
    m = mask.reshape(_ROWS, _COLS)

    yp_sc = yp[_TC_ROWS:].reshape(_SC_N)
    yt_sc = yt[_TC_ROWS:].reshape(_SC_N)
    m8_sc = m[_TC_ROWS:].view(jnp.uint8)

    ss, sc = _sc_partials(yp_sc, yt_sc, m8_sc)
    ts, tc = _tc_partials(yp[:_TC_ROWS], yt[:_TC_ROWS], m[:_TC_ROWS])

    return (ts[0, 0] + jnp.sum(ss)) / (tc[0, 0] + jnp.sum(sc))


# speedup vs baseline: 31.7625x; 1.9679x over previous
import jax
import jax.numpy as jnp
from jax import lax
from jax.experimental import pallas as pl
from jax.experimental.pallas import tpu as pltpu
from jax.experimental.pallas import tpu_sc as plsc

_ROWS = 8192
_COLS = 4096
_N = _ROWS * _COLS

_SC_ROWS = 2048
_TC_ROWS = _ROWS - _SC_ROWS
_BLOCK_ROWS = 256

_NW = 32
_WROWS = _SC_ROWS // _NW
_HALF = _COLS // 2
_NCH = (_WROWS // 8) * 2
_IT = 2 * (_HALF // 16)


def _tc_kernel(yp_ref, yt_ref, m_ref, sum_ref, cnt_ref, acc_s, acc_c):
    i = pl.program_id(0)

    @pl.when(i == 0)
    def _init():
        acc_s[...] = jnp.zeros_like(acc_s)
        acc_c[...] = jnp.zeros_like(acc_c)

    d = yp_ref[...] - yt_ref[...]
    m = m_ref[...]
    sq = jnp.where(m, d * d, jnp.float32(0.0))
    c = m.astype(jnp.float32)
    ps = sq[0:8]
    pc = c[0:8]
    for k in range(1, _BLOCK_ROWS // 8):
        ps = ps + sq[8 * k:8 * k + 8]
        pc = pc + c[8 * k:8 * k + 8]
    acc_s[...] += ps
    acc_c[...] += pc

    @pl.when(i == pl.num_programs(0) - 1)
    def _fini():
        sum_ref[0, 0] = jnp.sum(acc_s[...])
        cnt_ref[0, 0] = jnp.sum(acc_c[...])


def _tc_partials(yp, yt, m):
    grid = (_TC_ROWS // _BLOCK_ROWS,)
    in_spec = pl.BlockSpec((_BLOCK_ROWS, _COLS), lambda i: (i, 0))
    out_spec = pl.BlockSpec(memory_space=pltpu.SMEM)
    return pl.pallas_call(
        _tc_kernel,
        grid=grid,
        in_specs=[in_spec, in_spec, in_spec],
        out_specs=[out_spec, out_spec],
        out_shape=[
            jax.ShapeDtypeStruct((1, 1), jnp.float32),
            jax.ShapeDtypeStruct((1, 1), jnp.float32),
        ],
        scratch_shapes=[
            pltpu.VMEM((8, _COLS), jnp.float32),
            pltpu.VMEM((8, _COLS), jnp.float32),
        ],
    )(yp, yt, m)


def _sc_masked_mse(yp_hbm, yt_hbm, m_hbm, out_s_hbm, out_c_hbm,
                   ypb, ytb, mb, accv, cntv,
                   s_yp0, s_yp1, s_yt0, s_yt1, s_m0, s_m1):
    wid = lax.axis_index("s") * 2 + lax.axis_index("c")
    row0 = _TC_ROWS + wid * _WROWS

    fsems = ((s_yp0, s_yt0), (s_yp1, s_yt1))
    msems = (s_m0, s_m1)


    def fstart(c, b):
        band = 4 * (c >> 3) + (c & 3)
        half = (c >> 2) & 1
        r = pl.multiple_of(row0 + 8 * band, 8)
        co = pl.multiple_of(half * _HALF, _HALF)
        pltpu.async_copy(yp_hbm.at[pl.ds(r, 8), pl.ds(co, _HALF)],
                         ypb.at[b], fsems[b][0])
        pltpu.async_copy(yt_hbm.at[pl.ds(r, 8), pl.ds(co, _HALF)],
                         ytb.at[b], fsems[b][1])

    def fwait(c, b):
        band = 4 * (c >> 3) + (c & 3)
        half = (c >> 2) & 1
        r = pl.multiple_of(row0 + 8 * band, 8)
        co = pl.multiple_of(half * _HALF, _HALF)
        pltpu.make_async_copy(yp_hbm.at[pl.ds(r, 8), pl.ds(co, _HALF)],
                              ypb.at[b], fsems[b][0]).wait()
        pltpu.make_async_copy(yt_hbm.at[pl.ds(r, 8), pl.ds(co, _HALF)],
                              ytb.at[b], fsems[b][1]).wait()

    def mstart(mi):
        ms = mi & 1
        r = pl.multiple_of(row0 + 32 * (mi >> 1), 32)
        co = pl.multiple_of((mi & 1) * _HALF, _HALF)
        pltpu.async_copy(m_hbm.at[pl.ds(r, 32), pl.ds(co, _HALF)],
                         mb.at[ms], msems[ms])

    def mwait(mi):
        ms = mi & 1
        r = pl.multiple_of(row0 + 32 * (mi >> 1), 32)
        co = pl.multiple_of((mi & 1) * _HALF, _HALF)
        pltpu.make_async_copy(m_hbm.at[pl.ds(r, 32), pl.ds(co, _HALF)],
                              mb.at[ms], msems[ms]).wait()

    fstart(0, 0)
    fstart(1, 1)
    mstart(0)
    mstart(1)

    mbi = mb.bitcast(jnp.int32)

    def chunk_compute(c, b, accs):
        ms = (c >> 2) & 1
        mrow0 = (c & 3) * 2

        def body(g, carry):
            a0, a1, a2, a3, c0, c1, c2, c3 = carry
            rr = g >> 7
            co = pl.multiple_of((g & 127) * 16, 16)
            m32 = mbi[ms, mrow0 + rr, pl.ds(co, 16)]

            def term(j, a, cn):
                r = 4 * rr + j
                d = ypb[b, r, pl.ds(co, 16)] - ytb[b, r, pl.ds(co, 16)]
                mf = ((m32 >> (8 * j)) & 1).astype(jnp.float32)
                return a + d * d * mf, cn + mf

            a0, c0 = term(0, a0, c0)
            a1, c1 = term(1, a1, c1)
            a2, c2 = term(2, a2, c2)
            a3, c3 = term(3, a3, c3)
            return (a0, a1, a2, a3, c0, c1, c2, c3)

        return lax.fori_loop(0, _IT, body, accs, unroll=4)

    z = jnp.zeros((16,), jnp.float32)
    accs = (z, z, z, z, z, z, z, z)

    for c in range(_NCH):
        b = c & 1
        fwait(c, b)
        if c % 4 == 0:
            mwait(c >> 2)
        if c + 2 < _NCH:
            fstart(c + 2, b)
        if (c + 2) % 4 == 0 and 8 <= c + 2 < _NCH:
            mstart((c + 2) >> 2)
        accs = chunk_compute(c, b, accs)
    a0, a1, a2, a3, c0, c1, c2, c3 = accs
    accv[...] = (a0 + a1) + (a2 + a3)
    cntv[...] = (c0 + c1) + (c2 + c3)
    pltpu.sync_copy(accv, out_s_hbm.at[wid])
    pltpu.sync_copy(cntv, out_c_hbm.at[wid])


def _sc_partials(yp2, yt2, m2):
    mesh = plsc.VectorSubcoreMesh(core_axis_name="c", subcore_axis_name="s")
    f = pl.kernel(
        _sc_masked_mse,
        mesh=mesh,
        out_type=[
            jax.ShapeDtypeStruct((_NW, 16), jnp.float32),
            jax.ShapeDtypeStruct((_NW, 16), jnp.float32),
        ],
        scratch_types=[
            pltpu.VMEM((2, 8, _HALF), jnp.float32),
            pltpu.VMEM((2, 8, _HALF), jnp.float32),
            pltpu.VMEM((2, 32, _HALF), jnp.uint8),
            pltpu.VMEM((16,), jnp.float32),
            pltpu.VMEM((16,), jnp.float32),
            pltpu.SemaphoreType.DMA,
            pltpu.SemaphoreType.DMA,
            pltpu.SemaphoreType.DMA,
            pltpu.SemaphoreType.DMA,
            pltpu.SemaphoreType.DMA,
            pltpu.SemaphoreType.DMA,
        ],
        compiler_params=pltpu.CompilerParams(use_tc_tiling_on_sc=True),
    )
    return f(yp2, yt2, m2)


def kernel(y_pred, y_true, mask):
    yp = y_pred.reshape(_ROWS, _COLS)
    yt = y_true.reshape(_ROWS, _COLS)
    m = mask.reshape(_ROWS, _COLS)
    m8 = m.view(jnp.uint8)

    ss, sc = _sc_partials(yp, yt, m8)
    ts, tc = _tc_partials(yp, yt, m)

    return (ts[0, 0] + jnp.sum(ss)) / (tc[0, 0] + jnp.sum(sc))


# speedup vs baseline: 31.8014x; 1.0012x over previous
import jax
import jax.numpy as jnp
from jax import lax
from jax.experimental import pallas as pl
from jax.experimental.pallas import tpu as pltpu
from jax.experimental.pallas import tpu_sc as plsc

_ROWS = 8192
_COLS = 4096
_N = _ROWS * _COLS

_SC_ROWS = 3072
_TC_ROWS = _ROWS - _SC_ROWS
_BLOCK_ROWS = 256

_NW = 32
_WROWS = _SC_ROWS // _NW
_HALF = _COLS // 2
_NCH = (_WROWS // 8) * 2
_IT = 2 * (_HALF // 16)


def _tc_kernel(yp_ref, yt_ref, m_ref, sum_ref, cnt_ref, acc_s, acc_c):
    i = pl.program_id(0)

    @pl.when(i == 0)
    def _init():
        acc_s[...] = jnp.zeros_like(acc_s)
        acc_c[...] = jnp.zeros_like(acc_c)

    d = yp_ref[...] - yt_ref[...]
    m = m_ref[...]
    sq = jnp.where(m, d * d, jnp.float32(0.0))
    c = m.astype(jnp.float32)
    ps = sq[0:8]
    pc = c[0:8]
    for k in range(1, _BLOCK_ROWS // 8):
        ps = ps + sq[8 * k:8 * k + 8]
        pc = pc + c[8 * k:8 * k + 8]
    acc_s[...] += ps
    acc_c[...] += pc

    @pl.when(i == pl.num_programs(0) - 1)
    def _fini():
        sum_ref[0, 0] = jnp.sum(acc_s[...])
        cnt_ref[0, 0] = jnp.sum(acc_c[...])


def _tc_partials(yp, yt, m):
    grid = (_TC_ROWS // _BLOCK_ROWS,)
    in_spec = pl.BlockSpec((_BLOCK_ROWS, _COLS), lambda i: (i, 0))
    out_spec = pl.BlockSpec(memory_space=pltpu.SMEM)
    return pl.pallas_call(
        _tc_kernel,
        grid=grid,
        in_specs=[in_spec, in_spec, in_spec],
        out_specs=[out_spec, out_spec],
        out_shape=[
            jax.ShapeDtypeStruct((1, 1), jnp.float32),
            jax.ShapeDtypeStruct((1, 1), jnp.float32),
        ],
        scratch_shapes=[
            pltpu.VMEM((8, _COLS), jnp.float32),
            pltpu.VMEM((8, _COLS), jnp.float32),
        ],
    )(yp, yt, m)


def _sc_masked_mse(yp_hbm, yt_hbm, m_hbm, out_s_hbm, out_c_hbm,
                   ypb, ytb, mb, accv, cntv,
                   s_yp0, s_yp1, s_yt0, s_yt1, s_m0, s_m1):
    wid = lax.axis_index("s") * 2 + lax.axis_index("c")
    row0 = _TC_ROWS + wid * _WROWS

    fsems = ((s_yp0, s_yt0), (s_yp1, s_yt1))
    msems = (s_m0, s_m1)


    def fstart(c, b):
        band = 4 * (c >> 3) + (c & 3)
        half = (c >> 2) & 1
        r = pl.multiple_of(row0 + 8 * band, 8)
        co = pl.multiple_of(half * _HALF, _HALF)
        pltpu.async_copy(yp_hbm.at[pl.ds(r, 8), pl.ds(co, _HALF)],
                         ypb.at[b], fsems[b][0])
        pltpu.async_copy(yt_hbm.at[pl.ds(r, 8), pl.ds(co, _HALF)],
                         ytb.at[b], fsems[b][1])

    def fwait(c, b):
        band = 4 * (c >> 3) + (c & 3)
        half = (c >> 2) & 1
        r = pl.multiple_of(row0 + 8 * band, 8)
        co = pl.multiple_of(half * _HALF, _HALF)
        pltpu.make_async_copy(yp_hbm.at[pl.ds(r, 8), pl.ds(co, _HALF)],
                              ypb.at[b], fsems[b][0]).wait()
        pltpu.make_async_copy(yt_hbm.at[pl.ds(r, 8), pl.ds(co, _HALF)],
                              ytb.at[b], fsems[b][1]).wait()

    def mstart(mi):
        ms = mi & 1
        r = pl.multiple_of(wid * _WROWS + 32 * (mi >> 1), 32)
        co = pl.multiple_of((mi & 1) * _HALF, _HALF)
        pltpu.async_copy(m_hbm.at[pl.ds(r, 32), pl.ds(co, _HALF)],
                         mb.at[ms], msems[ms])

    def mwait(mi):
        ms = mi & 1
        r = pl.multiple_of(wid * _WROWS + 32 * (mi >> 1), 32)
        co = pl.multiple_of((mi & 1) * _HALF, _HALF)
        pltpu.make_async_copy(m_hbm.at[pl.ds(r, 32), pl.ds(co, _HALF)],
                              mb.at[ms], msems[ms]).wait()

    fstart(0, 0)
    fstart(1, 1)
    mstart(0)
    mstart(1)

    mbi = mb.bitcast(jnp.int32)

    def chunk_compute(c, b, accs):
        ms = (c >> 2) & 1
        mrow0 = (c & 3) * 2

        def body(g, carry):
            a0, a1, a2, a3, c0, c1, c2, c3 = carry
            rr = g >> 7
            co = pl.multiple_of((g & 127) * 16, 16)
            m32 = mbi[ms, mrow0 + rr, pl.ds(co, 16)]

            def term(j, a, cn):
                r = 4 * rr + j
                d = ypb[b, r, pl.ds(co, 16)] - ytb[b, r, pl.ds(co, 16)]
                mf = ((m32 >> (8 * j)) & 1).astype(jnp.float32)
                return a + d * d * mf, cn + mf

            a0, c0 = term(0, a0, c0)
            a1, c1 = term(1, a1, c1)
            a2, c2 = term(2, a2, c2)
            a3, c3 = term(3, a3, c3)
            return (a0, a1, a2, a3, c0, c1, c2, c3)

        return lax.fori_loop(0, _IT, body, accs, unroll=4)

    z = jnp.zeros((16,), jnp.float32)
    accs = (z, z, z, z, z, z, z, z)

    for c in range(_NCH):
        b = c & 1
        fwait(c, b)
        if c % 4 == 0:
            mwait(c >> 2)
        if c + 2 < _NCH:
            fstart(c + 2, b)
        if (c + 2) % 4 == 0 and 8 <= c + 2 < _NCH:
            mstart((c + 2) >> 2)
        accs = chunk_compute(c, b, accs)
    a0, a1, a2, a3, c0, c1, c2, c3 = accs
    accv[...] = (a0 + a1) + (a2 + a3)
    cntv[...] = (c0 + c1) + (c2 + c3)
    pltpu.sync_copy(accv, out_s_hbm.at[wid])
    pltpu.sync_copy(cntv, out_c_hbm.at[wid])


def _sc_partials(yp2, yt2, m2):
    mesh = plsc.VectorSubcoreMesh(core_axis_name="c", subcore_axis_name="s")
    f = pl.kernel(
        _sc_masked_mse,
        mesh=mesh,
        out_type=[
            jax.ShapeDtypeStruct((_NW, 16), jnp.float32),
            jax.ShapeDtypeStruct((_NW, 16), jnp.float32),
        ],
        scratch_types=[
            pltpu.VMEM((2, 8, _HALF), jnp.float32),
            pltpu.VMEM((2, 8, _HALF), jnp.float32),
            pltpu.VMEM((2, 32, _HALF), jnp.uint8),
            pltpu.VMEM((16,), jnp.float32),
            pltpu.VMEM((16,), jnp.float32),
            pltpu.SemaphoreType.DMA,
            pltpu.SemaphoreType.DMA,
            pltpu.SemaphoreType.DMA,
            pltpu.SemaphoreType.DMA,
            pltpu.SemaphoreType.DMA,
            pltpu.SemaphoreType.DMA,
        ],
        compiler_params=pltpu.CompilerParams(use_tc_tiling_on_sc=True),
    )
    return f(yp2, yt2, m2)


def kernel(y_pred, y_true, mask):
    yp = y_pred.reshape(_ROWS, _COLS)
    yt = y_true.reshape(_ROWS, _COLS)
    m = mask.reshape(_ROWS, _COLS)
    m8_sc = m[_TC_ROWS:].view(jnp.uint8)

    ss, sc = _sc_partials(yp, yt, m8_sc)
    ts, tc = _tc_partials(yp, yt, m)

    return (ts[0, 0] + jnp.sum(ss)) / (tc[0, 0] + jnp.sum(sc))


# speedup vs baseline: 40.3473x; 1.2687x over previous
import jax
import jax.numpy as jnp
from jax import lax
from jax.experimental import pallas as pl
from jax.experimental.pallas import tpu as pltpu
from jax.experimental.pallas import tpu_sc as plsc

_ROWS = 8192
_COLS = 4096
_N = _ROWS * _COLS

_SC_ROWS = 3072
_TC_ROWS = _ROWS - _SC_ROWS
_BLOCK_ROWS = 256

_NW = 32
_WROWS = _SC_ROWS // _NW
_HALF = _COLS // 2
_NCH = (_WROWS // 8) * 2
_IT = 2 * (_HALF // 16)


def _tc_kernel(yp_ref, yt_ref, m_ref, sum_ref, cnt_ref, acc_s, acc_c):
    i = pl.program_id(0)

    @pl.when(i == 0)
    def _init():
        acc_s[...] = jnp.zeros_like(acc_s)
        acc_c[...] = jnp.zeros_like(acc_c)

    d = yp_ref[...] - yt_ref[...]
    c = m_ref[...].astype(jnp.float32)
    sq = d * d * c
    ps = sq[0:8]
    pc = c[0:8]
    for k in range(1, _BLOCK_ROWS // 8):
        ps = ps + sq[8 * k:8 * k + 8]
        pc = pc + c[8 * k:8 * k + 8]
    acc_s[...] += ps
    acc_c[...] += pc

    @pl.when(i == pl.num_programs(0) - 1)
    def _fini():
        sum_ref[0, 0] = jnp.sum(acc_s[...])
        cnt_ref[0, 0] = jnp.sum(acc_c[...])


def _tc_partials(yp, yt, m):
    grid = (_TC_ROWS // _BLOCK_ROWS,)
    in_spec = pl.BlockSpec((_BLOCK_ROWS, _COLS), lambda i: (i, 0))
    out_spec = pl.BlockSpec(memory_space=pltpu.SMEM)
    return pl.pallas_call(
        _tc_kernel,
        grid=grid,
        in_specs=[in_spec, in_spec, in_spec],
        out_specs=[out_spec, out_spec],
        out_shape=[
            jax.ShapeDtypeStruct((1, 1), jnp.float32),
            jax.ShapeDtypeStruct((1, 1), jnp.float32),
        ],
        scratch_shapes=[
            pltpu.VMEM((8, _COLS), jnp.float32),
            pltpu.VMEM((8, _COLS), jnp.float32),
        ],
    )(yp, yt, m)


def _sc_masked_mse(yp_hbm, yt_hbm, m_hbm, out_s_hbm, out_c_hbm,
                   ypb, ytb, mb, accv, cntv,
                   s_yp0, s_yp1, s_yt0, s_yt1, s_m0, s_m1):
    wid = lax.axis_index("s") * 2 + lax.axis_index("c")
    row0 = _TC_ROWS + wid * _WROWS

    fsems = ((s_yp0, s_yt0), (s_yp1, s_yt1))
    msems = (s_m0, s_m1)


    def fstart(c, b):
        band = 4 * (c >> 3) + (c & 3)
        half = (c >> 2) & 1
        r = pl.multiple_of(row0 + 8 * band, 8)
        co = pl.multiple_of(half * _HALF, _HALF)
        pltpu.async_copy(yp_hbm.at[pl.ds(r, 8), pl.ds(co, _HALF)],
                         ypb.at[b], fsems[b][0])
        pltpu.async_copy(yt_hbm.at[pl.ds(r, 8), pl.ds(co, _HALF)],
                         ytb.at[b], fsems[b][1])

    def fwait(c, b):
        band = 4 * (c >> 3) + (c & 3)
        half = (c >> 2) & 1
        r = pl.multiple_of(row0 + 8 * band, 8)
        co = pl.multiple_of(half * _HALF, _HALF)
        pltpu.make_async_copy(yp_hbm.at[pl.ds(r, 8), pl.ds(co, _HALF)],
                              ypb.at[b], fsems[b][0]).wait()
        pltpu.make_async_copy(yt_hbm.at[pl.ds(r, 8), pl.ds(co, _HALF)],
                              ytb.at[b], fsems[b][1]).wait()

    def mstart(mi):
        ms = mi & 1
        r = pl.multiple_of(row0 + 32 * (mi >> 1), 32)
        co = pl.multiple_of((mi & 1) * _HALF, _HALF)
        pltpu.async_copy(m_hbm.at[pl.ds(r, 32), pl.ds(co, _HALF)],
                         mb.at[ms], msems[ms])

    def mwait(mi):
        ms = mi & 1
        r = pl.multiple_of(row0 + 32 * (mi >> 1), 32)
        co = pl.multiple_of((mi & 1) * _HALF, _HALF)
        pltpu.make_async_copy(m_hbm.at[pl.ds(r, 32), pl.ds(co, _HALF)],
                              mb.at[ms], msems[ms]).wait()

    fstart(0, 0)
    fstart(1, 1)
    mstart(0)
    mstart(1)

    mbi = mb.bitcast(jnp.int32)

    def chunk_compute(c, b, accs):
        ms = (c >> 2) & 1
        mrow0 = (c & 3) * 2

        def body(g, carry):
            a0, a1, a2, a3, c0, c1, c2, c3 = carry
            rr = g >> 7
            co = pl.multiple_of((g & 127) * 16, 16)
            m32 = mbi[ms, mrow0 + rr, pl.ds(co, 16)]

            def term(j, a, cn):
                r = 4 * rr + j
                d = ypb[b, r, pl.ds(co, 16)] - ytb[b, r, pl.ds(co, 16)]
                mf = ((m32 >> (8 * j)) & 1).astype(jnp.float32)
                return a + d * d * mf, cn + mf

            a0, c0 = term(0, a0, c0)
            a1, c1 = term(1, a1, c1)
            a2, c2 = term(2, a2, c2)
            a3, c3 = term(3, a3, c3)
            return (a0, a1, a2, a3, c0, c1, c2, c3)

        return lax.fori_loop(0, _IT, body, accs, unroll=4)

    z = jnp.zeros((16,), jnp.float32)
    accs = (z, z, z, z, z, z, z, z)

    for c in range(_NCH):
        b = c & 1
        fwait(c, b)
        if c % 4 == 0:
            mwait(c >> 2)
        if c + 2 < _NCH:
            fstart(c + 2, b)
        if (c + 2) % 4 == 0 and 8 <= c + 2 < _NCH:
            mstart((c + 2) >> 2)
        accs = chunk_compute(c, b, accs)
    a0, a1, a2, a3, c0, c1, c2, c3 = accs
    accv[...] = (a0 + a1) + (a2 + a3)
    cntv[...] = (c0 + c1) + (c2 + c3)
    pltpu.sync_copy(accv, out_s_hbm.at[wid])
    pltpu.sync_copy(cntv, out_c_hbm.at[wid])


def _sc_partials(yp2, yt2, m2):
    mesh = plsc.VectorSubcoreMesh(core_axis_name="c", subcore_axis_name="s")
    f = pl.kernel(
        _sc_masked_mse,
        mesh=mesh,
        out_type=[
            jax.ShapeDtypeStruct((_NW, 16), jnp.float32),
            jax.ShapeDtypeStruct((_NW, 16), jnp.float32),
        ],
        scratch_types=[
            pltpu.VMEM((2, 8, _HALF), jnp.float32),
            pltpu.VMEM((2, 8, _HALF), jnp.float32),
            pltpu.VMEM((2, 32, _HALF), jnp.uint8),
            pltpu.VMEM((16,), jnp.float32),
            pltpu.VMEM((16,), jnp.float32),
            pltpu.SemaphoreType.DMA,
            pltpu.SemaphoreType.DMA,
            pltpu.SemaphoreType.DMA,
            pltpu.SemaphoreType.DMA,
            pltpu.SemaphoreType.DMA,
            pltpu.SemaphoreType.DMA,
        ],
        compiler_params=pltpu.CompilerParams(use_tc_tiling_on_sc=True),
    )
    return f(yp2, yt2, m2)


def kernel(y_pred, y_true, mask):
    yp = y_pred.reshape(_ROWS, _COLS)
    yt = y_true.reshape(_ROWS, _COLS)
    m8 = mask.reshape(_ROWS, _COLS).view(jnp.uint8)

    ss, sc = _sc_partials(yp, yt, m8)
    ts, tc = _tc_partials(yp, yt, m8)

    return (ts[0, 0] + jnp.sum(ss)) / (tc[0, 0] + jnp.sum(sc))


# speedup vs baseline: 46.4654x; 1.1516x over previous
import jax
import jax.numpy as jnp
from jax import lax
from jax.experimental import pallas as pl
from jax.experimental.pallas import tpu as pltpu
from jax.experimental.pallas import tpu_sc as plsc

_ROWS = 8192
_COLS = 4096
_N = _ROWS * _COLS

_SC_ROWS = 3072
_TC_ROWS = _ROWS - _SC_ROWS
_BLOCK_ROWS = 256

_NW = 32
_WROWS = _SC_ROWS // _NW
_HALF = _COLS // 2
_NCH = (_WROWS // 8) * 2
_IT = 2 * (_HALF // 16)


def _tc_kernel(yp_ref, yt_ref, m_ref, sum_ref, cnt_ref, acc_s, acc_c):
    i = pl.program_id(0)

    @pl.when(i == 0)
    def _init():
        acc_s[...] = jnp.zeros_like(acc_s)
        acc_c[...] = jnp.zeros_like(acc_c)

    d = yp_ref[...] - yt_ref[...]
    c = m_ref[...].astype(jnp.float32)
    sq = d * d * c
    ps = sq[0:8]
    pc = c[0:8]
    for k in range(1, _BLOCK_ROWS // 8):
        ps = ps + sq[8 * k:8 * k + 8]
        pc = pc + c[8 * k:8 * k + 8]
    acc_s[...] += ps
    acc_c[...] += pc

    @pl.when(i == pl.num_programs(0) - 1)
    def _fini():
        sum_ref[0, 0] = jnp.sum(acc_s[...])
        cnt_ref[0, 0] = jnp.sum(acc_c[...])


def _tc_partials(yp, yt, m):
    grid = (_TC_ROWS // _BLOCK_ROWS,)
    in_spec = pl.BlockSpec((_BLOCK_ROWS, _COLS), lambda i: (i, 0))
    out_spec = pl.BlockSpec(memory_space=pltpu.SMEM)
    return pl.pallas_call(
        _tc_kernel,
        grid=grid,
        in_specs=[in_spec, in_spec, in_spec],
        out_specs=[out_spec, out_spec],
        out_shape=[
            jax.ShapeDtypeStruct((1, 1), jnp.float32),
            jax.ShapeDtypeStruct((1, 1), jnp.float32),
        ],
        scratch_shapes=[
            pltpu.VMEM((8, _COLS), jnp.float32),
            pltpu.VMEM((8, _COLS), jnp.float32),
        ],
    )(yp, yt, m)


def _sc_masked_mse(yp_hbm, yt_hbm, m_hbm, out_s_hbm, out_c_hbm,
                   ypb, ytb, mb, accv, cntv,
                   s_yp0, s_yp1, s_yt0, s_yt1, s_m0, s_m1):
    wid = lax.axis_index("s") * 2 + lax.axis_index("c")
    row0 = _TC_ROWS + wid * _WROWS

    fsems = ((s_yp0, s_yt0), (s_yp1, s_yt1))
    msems = (s_m0, s_m1)


    def fstart(c, b):
        band = 4 * (c >> 3) + (c & 3)
        half = (c >> 2) & 1
        r = pl.multiple_of(row0 + 8 * band, 8)
        co = pl.multiple_of(half * _HALF, _HALF)
        pltpu.async_copy(yp_hbm.at[pl.ds(r, 8), pl.ds(co, _HALF)],
                         ypb.at[b], fsems[b][0])
        pltpu.async_copy(yt_hbm.at[pl.ds(r, 8), pl.ds(co, _HALF)],
                         ytb.at[b], fsems[b][1])

    def fwait(c, b):
        band = 4 * (c >> 3) + (c & 3)
        half = (c >> 2) & 1
        r = pl.multiple_of(row0 + 8 * band, 8)
        co = pl.multiple_of(half * _HALF, _HALF)
        pltpu.make_async_copy(yp_hbm.at[pl.ds(r, 8), pl.ds(co, _HALF)],
                              ypb.at[b], fsems[b][0]).wait()
        pltpu.make_async_copy(yt_hbm.at[pl.ds(r, 8), pl.ds(co, _HALF)],
                              ytb.at[b], fsems[b][1]).wait()

    def mstart(mi):
        ms = mi & 1
        r = pl.multiple_of(wid * _WROWS + 32 * (mi >> 1), 32)
        co = pl.multiple_of((mi & 1) * _HALF, _HALF)
        pltpu.async_copy(m_hbm.at[pl.ds(r, 32), pl.ds(co, _HALF)],
                         mb.at[ms], msems[ms])

    def mwait(mi):
        ms = mi & 1
        r = pl.multiple_of(wid * _WROWS + 32 * (mi >> 1), 32)
        co = pl.multiple_of((mi & 1) * _HALF, _HALF)
        pltpu.make_async_copy(m_hbm.at[pl.ds(r, 32), pl.ds(co, _HALF)],
                              mb.at[ms], msems[ms]).wait()

    fstart(0, 0)
    fstart(1, 1)
    mstart(0)
    mstart(1)

    mbi = mb.bitcast(jnp.int32)

    def chunk_compute(c, b, accs):
        ms = (c >> 2) & 1
        mrow0 = (c & 3) * 2

        def body(g, carry):
            a0, a1, a2, a3, c0, c1, c2, c3 = carry
            rr = g >> 7
            co = pl.multiple_of((g & 127) * 16, 16)
            m32 = mbi[ms, mrow0 + rr, pl.ds(co, 16)]

            def term(j, a, cn):
                r = 4 * rr + j
                d = ypb[b, r, pl.ds(co, 16)] - ytb[b, r, pl.ds(co, 16)]
                mf = ((m32 >> (8 * j)) & 1).astype(jnp.float32)
                return a + d * d * mf, cn + mf

            a0, c0 = term(0, a0, c0)
            a1, c1 = term(1, a1, c1)
            a2, c2 = term(2, a2, c2)
            a3, c3 = term(3, a3, c3)
            return (a0, a1, a2, a3, c0, c1, c2, c3)

        return lax.fori_loop(0, _IT, body, accs, unroll=4)

    z = jnp.zeros((16,), jnp.float32)
    accs = (z, z, z, z, z, z, z, z)

    for c in range(_NCH):
        b = c & 1
        fwait(c, b)
        if c % 4 == 0:
            mwait(c >> 2)
        if c + 2 < _NCH:
            fstart(c + 2, b)
        if (c + 2) % 4 == 0 and 8 <= c + 2 < _NCH:
            mstart((c + 2) >> 2)
        accs = chunk_compute(c, b, accs)
    a0, a1, a2, a3, c0, c1, c2, c3 = accs
    accv[...] = (a0 + a1) + (a2 + a3)
    cntv[...] = (c0 + c1) + (c2 + c3)
    pltpu.sync_copy(accv, out_s_hbm.at[wid])
    pltpu.sync_copy(cntv, out_c_hbm.at[wid])


def _sc_partials(yp2, yt2, m2):
    mesh = plsc.VectorSubcoreMesh(core_axis_name="c", subcore_axis_name="s")
    f = pl.kernel(
        _sc_masked_mse,
        mesh=mesh,
        out_type=[
            jax.ShapeDtypeStruct((_NW, 16), jnp.float32),
            jax.ShapeDtypeStruct((_NW, 16), jnp.float32),
        ],
        scratch_types=[
            pltpu.VMEM((2, 8, _HALF), jnp.float32),
            pltpu.VMEM((2, 8, _HALF), jnp.float32),
            pltpu.VMEM((2, 32, _HALF), jnp.uint8),
            pltpu.VMEM((16,), jnp.float32),
            pltpu.VMEM((16,), jnp.float32),
            pltpu.SemaphoreType.DMA,
            pltpu.SemaphoreType.DMA,
            pltpu.SemaphoreType.DMA,
            pltpu.SemaphoreType.DMA,
            pltpu.SemaphoreType.DMA,
            pltpu.SemaphoreType.DMA,
        ],
        compiler_params=pltpu.CompilerParams(use_tc_tiling_on_sc=True),
    )
    return f(yp2, yt2, m2)


def kernel(y_pred, y_true, mask):
    yp = y_pred.reshape(_ROWS, _COLS)
    yt = y_true.reshape(_ROWS, _COLS)
    m = mask.reshape(_ROWS, _COLS)
    m8_tc = m[:_TC_ROWS].view(jnp.uint8)
    m8_sc = m[_TC_ROWS:].view(jnp.uint8)

    ss, sc = _sc_partials(yp, yt, m8_sc)
    ts, tc = _tc_partials(yp, yt, m8_tc)

    return (ts[0, 0] + jnp.sum(ss)) / (tc[0, 0] + jnp.sum(sc))


# speedup vs baseline: 46.7084x; 1.0052x over previous
import jax
import jax.numpy as jnp
from jax import lax
from jax.experimental import pallas as pl
from jax.experimental.pallas import tpu as pltpu
from jax.experimental.pallas import tpu_sc as plsc

_ROWS = 8192
_COLS = 4096
_N = _ROWS * _COLS

_SC_ROWS = 4096
_TC_ROWS = _ROWS - _SC_ROWS
_BLOCK_ROWS = 512

_NW = 32
_WROWS = _SC_ROWS // _NW
_HALF = _COLS // 2
_NCH = (_WROWS // 8) * 2
_IT = 2 * (_HALF // 16)


def _tc_kernel(yp_ref, yt_ref, m_ref, sum_ref, cnt_ref, acc_s, acc_c):
    i = pl.program_id(0)

    @pl.when(i == 0)
    def _init():
        acc_s[...] = jnp.zeros_like(acc_s)
        acc_c[...] = jnp.zeros_like(acc_c)

    d = yp_ref[...] - yt_ref[...]
    c = m_ref[...].astype(jnp.float32)
    sq = d * d * c
    ps = sq[0:8]
    pc = c[0:8]
    for k in range(1, _BLOCK_ROWS // 8):
        ps = ps + sq[8 * k:8 * k + 8]
        pc = pc + c[8 * k:8 * k + 8]
    acc_s[...] += ps
    acc_c[...] += pc

    @pl.when(i == pl.num_programs(0) - 1)
    def _fini():
        sum_ref[0, 0] = jnp.sum(acc_s[...])
        cnt_ref[0, 0] = jnp.sum(acc_c[...])


def _tc_partials(yp, yt, m):
    grid = (_TC_ROWS // _BLOCK_ROWS,)
    in_spec = pl.BlockSpec((_BLOCK_ROWS, _COLS), lambda i: (i, 0))
    out_spec = pl.BlockSpec(memory_space=pltpu.SMEM)
    return pl.pallas_call(
        _tc_kernel,
        grid=grid,
        in_specs=[in_spec, in_spec, in_spec],
        out_specs=[out_spec, out_spec],
        out_shape=[
            jax.ShapeDtypeStruct((1, 1), jnp.float32),
            jax.ShapeDtypeStruct((1, 1), jnp.float32),
        ],
        scratch_shapes=[
            pltpu.VMEM((8, _COLS), jnp.float32),
            pltpu.VMEM((8, _COLS), jnp.float32),
        ],
    )(yp, yt, m)


def _sc_masked_mse(yp_hbm, yt_hbm, m_hbm, out_s_hbm, out_c_hbm,
                   ypb, ytb, mb, accv, cntv,
                   s_yp0, s_yp1, s_yt0, s_yt1, s_m0, s_m1):
    wid = lax.axis_index("s") * 2 + lax.axis_index("c")
    row0 = _TC_ROWS + wid * _WROWS

    fsems = ((s_yp0, s_yt0), (s_yp1, s_yt1))
    msems = (s_m0, s_m1)


    def fstart(c, b):
        band = 4 * (c >> 3) + (c & 3)
        half = (c >> 2) & 1
        r = pl.multiple_of(row0 + 8 * band, 8)
        co = pl.multiple_of(half * _HALF, _HALF)
        pltpu.async_copy(yp_hbm.at[pl.ds(r, 8), pl.ds(co, _HALF)],
                         ypb.at[b], fsems[b][0])
        pltpu.async_copy(yt_hbm.at[pl.ds(r, 8), pl.ds(co, _HALF)],
                         ytb.at[b], fsems[b][1])

    def fwait(c, b):
        band = 4 * (c >> 3) + (c & 3)
        half = (c >> 2) & 1
        r = pl.multiple_of(row0 + 8 * band, 8)
        co = pl.multiple_of(half * _HALF, _HALF)
        pltpu.make_async_copy(yp_hbm.at[pl.ds(r, 8), pl.ds(co, _HALF)],
                              ypb.at[b], fsems[b][0]).wait()
        pltpu.make_async_copy(yt_hbm.at[pl.ds(r, 8), pl.ds(co, _HALF)],
                              ytb.at[b], fsems[b][1]).wait()

    def mstart(mi):
        ms = mi & 1
        r = pl.multiple_of(wid * _WROWS + 32 * (mi >> 1), 32)
        co = pl.multiple_of((mi & 1) * _HALF, _HALF)
        pltpu.async_copy(m_hbm.at[pl.ds(r, 32), pl.ds(co, _HALF)],
                         mb.at[ms], msems[ms])

    def mwait(mi):
        ms = mi & 1
        r = pl.multiple_of(wid * _WROWS + 32 * (mi >> 1), 32)
        co = pl.multiple_of((mi & 1) * _HALF, _HALF)
        pltpu.make_async_copy(m_hbm.at[pl.ds(r, 32), pl.ds(co, _HALF)],
                              mb.at[ms], msems[ms]).wait()

    fstart(0, 0)
    fstart(1, 1)
    mstart(0)
    mstart(1)

    mbi = mb.bitcast(jnp.int32)

    def chunk_compute(c, b, accs):
        ms = (c >> 2) & 1
        mrow0 = (c & 3) * 2

        def body(g, carry):
            a0, a1, a2, a3, c0, c1, c2, c3 = carry
            rr = g >> 7
            co = pl.multiple_of((g & 127) * 16, 16)
            m32 = mbi[ms, mrow0 + rr, pl.ds(co, 16)]

            def term(j, a, cn):
                r = 4 * rr + j
                d = ypb[b, r, pl.ds(co, 16)] - ytb[b, r, pl.ds(co, 16)]
                mf = ((m32 >> (8 * j)) & 1).astype(jnp.float32)
                return a + d * d * mf, cn + mf

            a0, c0 = term(0, a0, c0)
            a1, c1 = term(1, a1, c1)
            a2, c2 = term(2, a2, c2)
            a3, c3 = term(3, a3, c3)
            return (a0, a1, a2, a3, c0, c1, c2, c3)

        return lax.fori_loop(0, _IT, body, accs, unroll=4)

    z = jnp.zeros((16,), jnp.float32)
    accs = (z, z, z, z, z, z, z, z)

    for c in range(_NCH):
        b = c & 1
        fwait(c, b)
        if c % 4 == 0:
            mwait(c >> 2)
        if c + 2 < _NCH:
            fstart(c + 2, b)
        if (c + 2) % 4 == 0 and 8 <= c + 2 < _NCH:
            mstart((c + 2) >> 2)
        accs = chunk_compute(c, b, accs)
    a0, a1, a2, a3, c0, c1, c2, c3 = accs
    accv[...] = (a0 + a1) + (a2 + a3)
    cntv[...] = (c0 + c1) + (c2 + c3)
    pltpu.sync_copy(accv, out_s_hbm.at[wid])
    pltpu.sync_copy(cntv, out_c_hbm.at[wid])


def _sc_partials(yp2, yt2, m2):
    mesh = plsc.VectorSubcoreMesh(core_axis_name="c", subcore_axis_name="s")
    f = pl.kernel(
        _sc_masked_mse,
        mesh=mesh,
        out_type=[
            jax.ShapeDtypeStruct((_NW, 16), jnp.float32),
            jax.ShapeDtypeStruct((_NW, 16), jnp.float32),
        ],
        scratch_types=[
            pltpu.VMEM((2, 8, _HALF), jnp.float32),
            pltpu.VMEM((2, 8, _HALF), jnp.float32),
            pltpu.VMEM((2, 32, _HALF), jnp.uint8),
            pltpu.VMEM((16,), jnp.float32),
            pltpu.VMEM((16,), jnp.float32),
            pltpu.SemaphoreType.DMA,
            pltpu.SemaphoreType.DMA,
            pltpu.SemaphoreType.DMA,
            pltpu.SemaphoreType.DMA,
            pltpu.SemaphoreType.DMA,
            pltpu.SemaphoreType.DMA,
        ],
        compiler_params=pltpu.CompilerParams(use_tc_tiling_on_sc=True),
    )
    return f(yp2, yt2, m2)


def kernel(y_pred, y_true, mask):
    yp = y_pred.reshape(_ROWS, _COLS)
    yt = y_true.reshape(_ROWS, _COLS)
    m = mask.reshape(_ROWS, _COLS)
    m8_tc = m[:_TC_ROWS].view(jnp.uint8)
    m8_sc = m[_TC_ROWS:].view(jnp.uint8)

    ss, sc = _sc_partials(yp, yt, m8_sc)
    ts, tc = _tc_partials(yp, yt, m8_tc)

    return (ts[0, 0] + jnp.sum(ss)) / (tc[0, 0] + jnp.sum(sc))
